# baseline (speedup 1.0000x reference)
import functools

import jax
import jax.numpy as jnp
from jax import lax
from jax.experimental import pallas as pl
from jax.experimental.pallas import tpu as pltpu
from jax.experimental.pallas import tpu_sc as plsc

B = 32
POOL = 1000
CTX_LEN = 36
CTX_DIM = 512
N_CLS = 100
SUF_LEN = 40
SEQ_LEN = 77


def _gather_body(idx_ref, ent_ref, out_ref):
    out_ref[...] = ent_ref[...]


def _build_body(nl_ref, prefix_ref, suffix_ref, ctx_ref, out_ref):
    c = pl.program_id(0)
    nl = nl_ref[c]
    s = suffix_ref[0]
    p = prefix_ref[0]

    del nl, s, p
    out_ref[...] = jnp.zeros((B, SEQ_LEN, CTX_DIM), jnp.float32)


def _tok_body(tok_ref, out_ref):
    out_ref[...] = tok_ref[...][None]


def kernel(indexs, entity_prompts, name_lens, token_prefix, token_suffix,
           tokenized_prompts, current_task):
    indexs = indexs.astype(jnp.int32)
    name_lens = name_lens.astype(jnp.int32)

    ctx = pl.pallas_call(
        _gather_body,
        grid_spec=pltpu.PrefetchScalarGridSpec(
            num_scalar_prefetch=1,
            grid=(B,),
            in_specs=[
                pl.BlockSpec((1, CTX_LEN, CTX_DIM),
                             lambda b, idx: (idx[b], 0, 0)),
            ],
            out_specs=pl.BlockSpec((1, CTX_LEN, CTX_DIM),
                                   lambda b, idx: (b, 0, 0)),
        ),
        out_shape=jax.ShapeDtypeStruct((B, CTX_LEN, CTX_DIM), jnp.float32),
    )(indexs, entity_prompts)

    def _sc_probe_body(idx_hbm, out_hbm, buf, sem):
        cid = lax.axis_index("c")
        sid = lax.axis_index("s")
        w = sid * 2 + cid
        base = w * 100
        copies = [
            pltpu.async_copy(buf, out_hbm.at[pl.ds(base + 2 * i, 2)], sem)
            for i in range(50)
        ]
        for c in copies:
            c.wait()

    mesh = plsc.VectorSubcoreMesh(core_axis_name="c", subcore_axis_name="s")
    dummy = pl.kernel(
        _sc_probe_body,
        out_type=jax.ShapeDtypeStruct((N_CLS * B, SEQ_LEN, CTX_DIM),
                                      jnp.float32),
        mesh=mesh,
        scratch_types=[pltpu.VMEM((2, SEQ_LEN, CTX_DIM), jnp.float32),
                       pltpu.SemaphoreType.DMA],
    )(indexs)

    prompts = pl.pallas_call(
        _build_body,
        grid_spec=pltpu.PrefetchScalarGridSpec(
            num_scalar_prefetch=1,
            grid=(N_CLS,),
            in_specs=[
                pl.BlockSpec((1, 1, CTX_DIM), lambda c, nl: (c, 0, 0)),
                pl.BlockSpec((1, SUF_LEN, CTX_DIM), lambda c, nl: (c, 0, 0)),
                pl.BlockSpec((B, CTX_LEN, CTX_DIM), lambda c, nl: (0, 0, 0)),
            ],
            out_specs=pl.BlockSpec((B, SEQ_LEN, CTX_DIM),
                                   lambda c, nl: (c, 0, 0)),
        ),
        out_shape=jax.ShapeDtypeStruct((N_CLS * B, SEQ_LEN, CTX_DIM),
                                       jnp.float32),
    )(name_lens, token_prefix, token_suffix, ctx)
    dep = (dummy[0, 0, 0] * 0.0).astype(jnp.int32)

    tok = pl.pallas_call(
        _tok_body,
        grid=(B,),
        in_specs=[pl.BlockSpec((N_CLS, SEQ_LEN), lambda b: (0, 0))],
        out_specs=pl.BlockSpec((1, N_CLS, SEQ_LEN), lambda b: (b, 0, 0)),
        out_shape=jax.ShapeDtypeStruct((B, N_CLS, SEQ_LEN),
                                       tokenized_prompts.dtype),
    )(tokenized_prompts)

    return (prompts, tok.reshape(B * N_CLS, SEQ_LEN) + dep)


# speedup vs baseline: 1.0813x; 1.0813x over previous
import jax
import jax.numpy as jnp
from jax import lax
from jax.experimental import pallas as pl
from jax.experimental.pallas import tpu as pltpu
from jax.experimental.pallas import tpu_sc as plsc

B = 32
POOL = 1000
CTX_LEN = 36
CTX_DIM = 512
N_CLS = 100
SUF_LEN = 40
SEQ_LEN = 77
LANES = 16


def _sc_gather_body(idx_hbm, entity_hbm, out_hbm, idx_v, ctx_v, sem):
    cid = lax.axis_index("c")
    sid = lax.axis_index("s")
    w = sid * 2 + cid
    pltpu.sync_copy(idx_hbm, idx_v.at[pl.ds(0, B)])
    my_idx = idx_v[pl.ds(w, LANES)][0]
    pltpu.async_copy(entity_hbm.at[my_idx], ctx_v, sem).wait()
    pltpu.sync_copy(ctx_v, out_hbm.at[w])


def _build_body(nl_ref, prefix_ref, suffix_ref, ctx_ref, out_ref):
    c = pl.program_id(0)
    nl = nl_ref[c]
    s = suffix_ref[0]
    p = prefix_ref[0]

    def _emit(k):
        def br():
            head = (p if k == 0
                    else jnp.concatenate([p, s[:k]], axis=0))
            out_ref[:, :k + 1, :] = jnp.broadcast_to(
                head[None], (B, k + 1, CTX_DIM))
            out_ref[:, k + 1:k + 1 + CTX_LEN, :] = ctx_ref[...]
            out_ref[:, k + 1 + CTX_LEN:, :] = jnp.broadcast_to(
                s[None, k:], (B, SUF_LEN - k, CTX_DIM))
        return br

    jax.lax.switch(nl, [_emit(k) for k in range(20)])


def _tok_body(tok_ref, out_ref):
    out_ref[...] = tok_ref[...][None]


def kernel(indexs, entity_prompts, name_lens, token_prefix, token_suffix,
           tokenized_prompts, current_task):
    indexs = indexs.astype(jnp.int32)
    name_lens = name_lens.astype(jnp.int32)
    entity4 = entity_prompts.reshape(POOL, CTX_LEN, 4, 128)

    mesh = plsc.VectorSubcoreMesh(core_axis_name="c", subcore_axis_name="s")
    ctx4 = pl.kernel(
        _sc_gather_body,
        out_type=jax.ShapeDtypeStruct((B, CTX_LEN, 4, 128), jnp.float32),
        mesh=mesh,
        scratch_types=[
            pltpu.VMEM((B + LANES,), jnp.int32),
            pltpu.VMEM((CTX_LEN, 4, 128), jnp.float32),
            pltpu.SemaphoreType.DMA,
        ],
    )(indexs, entity4)
    ctx = ctx4.reshape(B, CTX_LEN, CTX_DIM)

    prompts = pl.pallas_call(
        _build_body,
        grid_spec=pltpu.PrefetchScalarGridSpec(
            num_scalar_prefetch=1,
            grid=(N_CLS,),
            in_specs=[
                pl.BlockSpec((1, 1, CTX_DIM), lambda c, nl: (c, 0, 0)),
                pl.BlockSpec((1, SUF_LEN, CTX_DIM), lambda c, nl: (c, 0, 0)),
                pl.BlockSpec((B, CTX_LEN, CTX_DIM), lambda c, nl: (0, 0, 0)),
            ],
            out_specs=pl.BlockSpec((B, SEQ_LEN, CTX_DIM),
                                   lambda c, nl: (c, 0, 0)),
        ),
        out_shape=jax.ShapeDtypeStruct((N_CLS * B, SEQ_LEN, CTX_DIM),
                                       jnp.float32),
    )(name_lens, token_prefix, token_suffix, ctx)

    tok = pl.pallas_call(
        _tok_body,
        grid=(B,),
        in_specs=[pl.BlockSpec((N_CLS, SEQ_LEN), lambda b: (0, 0))],
        out_specs=pl.BlockSpec((1, N_CLS, SEQ_LEN), lambda b: (b, 0, 0)),
        out_shape=jax.ShapeDtypeStruct((B, N_CLS, SEQ_LEN),
                                       tokenized_prompts.dtype),
    )(tokenized_prompts)

    return (prompts, tok.reshape(B * N_CLS, SEQ_LEN))


# speedup vs baseline: 1.3780x; 1.2743x over previous
import jax
import jax.numpy as jnp
from jax import lax
from jax.experimental import pallas as pl
from jax.experimental.pallas import tpu as pltpu
from jax.experimental.pallas import tpu_sc as plsc

B = 32
POOL = 1000
CTX_LEN = 36
CTX_DIM = 512
N_CLS = 100
SUF_LEN = 40
SEQ_LEN = 77
LANES = 16


def _sc_gather_body(idx_hbm, entity_hbm, out_hbm, idx_v, buf_v, sem):
    cid = lax.axis_index("c")
    sid = lax.axis_index("s")
    w = sid * 2 + cid
    pltpu.sync_copy(idx_hbm, idx_v.at[pl.ds(0, B)])
    my_idx = idx_v[pl.ds(w, LANES)][0]
    g = pl.multiple_of((my_idx // 8) * 8, 8)
    r = my_idx - g
    for off, n in ((0, 16), (16, 16), (32, 4)):
        n_pad = 8 if n == 4 else n
        pltpu.sync_copy(entity_hbm.at[pl.ds(off, n), pl.ds(g, 8)],
                        buf_v.at[pl.ds(0, n)])
        pltpu.sync_copy(buf_v.at[pl.ds(0, n_pad), r],
                        out_hbm.at[w, pl.ds(off, n_pad)])


def _build_body(nl_ref, prefix_ref, suffix_ref, ctx_ref, out_ref):
    c = pl.program_id(0)
    nl = nl_ref[c]
    s = suffix_ref[0]
    p = prefix_ref[0]

    def _emit(k):
        def br():
            head = (p if k == 0
                    else jnp.concatenate([p, s[:k]], axis=0))
            out_ref[:, :k + 1, :] = jnp.broadcast_to(
                head[None], (B, k + 1, CTX_DIM))
            out_ref[:, k + 1:k + 1 + CTX_LEN, :] = ctx_ref[:, :CTX_LEN, :]
            out_ref[:, k + 1 + CTX_LEN:, :] = jnp.broadcast_to(
                s[None, k:], (B, SUF_LEN - k, CTX_DIM))
        return br

    jax.lax.switch(nl, [_emit(k) for k in range(20)])


def _tok_body(tok_ref, out_ref):
    out_ref[...] = tok_ref[...][None]


def kernel(indexs, entity_prompts, name_lens, token_prefix, token_suffix,
           tokenized_prompts, current_task):
    indexs = indexs.astype(jnp.int32)
    name_lens = name_lens.astype(jnp.int32)
    entity_t = jnp.transpose(entity_prompts, (1, 0, 2))

    mesh = plsc.VectorSubcoreMesh(core_axis_name="c", subcore_axis_name="s")
    ctx = pl.kernel(
        _sc_gather_body,
        out_type=jax.ShapeDtypeStruct((B, SUF_LEN, CTX_DIM), jnp.float32),
        mesh=mesh,
        scratch_types=[
            pltpu.VMEM((B + LANES,), jnp.int32),
            pltpu.VMEM((LANES, 8, CTX_DIM), jnp.float32),
            pltpu.SemaphoreType.DMA,
        ],
    )(indexs, entity_t)

    prompts = pl.pallas_call(
        _build_body,
        grid_spec=pltpu.PrefetchScalarGridSpec(
            num_scalar_prefetch=1,
            grid=(N_CLS,),
            in_specs=[
                pl.BlockSpec((1, 1, CTX_DIM), lambda c, nl: (c, 0, 0)),
                pl.BlockSpec((1, SUF_LEN, CTX_DIM), lambda c, nl: (c, 0, 0)),
                pl.BlockSpec((B, SUF_LEN, CTX_DIM), lambda c, nl: (0, 0, 0)),
            ],
            out_specs=pl.BlockSpec((B, SEQ_LEN, CTX_DIM),
                                   lambda c, nl: (c, 0, 0)),
        ),
        out_shape=jax.ShapeDtypeStruct((N_CLS * B, SEQ_LEN, CTX_DIM),
                                       jnp.float32),
    )(name_lens, token_prefix, token_suffix, ctx)

    tok = pl.pallas_call(
        _tok_body,
        grid=(B,),
        in_specs=[pl.BlockSpec((N_CLS, SEQ_LEN), lambda b: (0, 0))],
        out_specs=pl.BlockSpec((1, N_CLS, SEQ_LEN), lambda b: (b, 0, 0)),
        out_shape=jax.ShapeDtypeStruct((B, N_CLS, SEQ_LEN),
                                       tokenized_prompts.dtype),
    )(tokenized_prompts)

    return (prompts, tok.reshape(B * N_CLS, SEQ_LEN))


# speedup vs baseline: 3.5601x; 2.5836x over previous
import jax
import jax.numpy as jnp
from jax import lax
from jax.experimental import pallas as pl
from jax.experimental.pallas import tpu as pltpu
from jax.experimental.pallas import tpu_sc as plsc

B = 32
POOL = 1000
CTX_LEN = 36
CTX_DIM = 512
N_CLS = 100
SUF_LEN = 40
SEQ_LEN = 77
LANES = 16


def _sc_gather_body(idx_hbm, entity_hbm, out_hbm, idx_v, buf_v, sem):
    cid = lax.axis_index("c")
    sid = lax.axis_index("s")
    w = sid * 2 + cid
    pltpu.sync_copy(idx_hbm, idx_v.at[pl.ds(0, B)])
    my_idx = idx_v[pl.ds(w, LANES)][0]
    g = pl.multiple_of((my_idx // 8) * 8, 8)
    r = my_idx - g
    for off, n in ((0, 16), (16, 16), (32, 4)):
        n_pad = 8 if n == 4 else n
        pltpu.sync_copy(entity_hbm.at[pl.ds(off, n), pl.ds(g, 8)],
                        buf_v.at[pl.ds(0, n)])
        pltpu.sync_copy(buf_v.at[pl.ds(0, n_pad), r],
                        out_hbm.at[w, pl.ds(off, n_pad)])


def _build_body(nl_ref, prefix_ref, suffix_ref, ctx_ref, out_ref):
    c = pl.program_id(0)
    nl = nl_ref[c]
    s = suffix_ref[0]
    p = prefix_ref[0]
    s1 = jnp.concatenate([p, s, s[:SEQ_LEN - SUF_LEN - 1]], axis=0)
    s2 = jnp.concatenate([s[:SEQ_LEN - SUF_LEN], s], axis=0)
    pos = jax.lax.broadcasted_iota(jnp.int32, (SEQ_LEN, CTX_DIM), 0)
    base = jnp.where(pos <= nl, s1, s2)
    out_ref[...] = jnp.broadcast_to(base[:, None, :], (SEQ_LEN, B, CTX_DIM))
    out_ref[pl.ds(nl + 1, CTX_LEN), :, :] = ctx_ref[...]


def _tok_body(tok_ref, out_ref):
    out_ref[...] = tok_ref[...][None]


def kernel(indexs, entity_prompts, name_lens, token_prefix, token_suffix,
           tokenized_prompts, current_task):
    indexs = indexs.astype(jnp.int32)
    name_lens = name_lens.astype(jnp.int32)
    entity_t = jnp.transpose(entity_prompts, (1, 0, 2))

    mesh = plsc.VectorSubcoreMesh(core_axis_name="c", subcore_axis_name="s")
    ctx = pl.kernel(
        _sc_gather_body,
        out_type=jax.ShapeDtypeStruct((B, SUF_LEN, CTX_DIM), jnp.float32),
        mesh=mesh,
        scratch_types=[
            pltpu.VMEM((B + LANES,), jnp.int32),
            pltpu.VMEM((LANES, 8, CTX_DIM), jnp.float32),
            pltpu.SemaphoreType.DMA,
        ],
    )(indexs, entity_t)

    ctx_t = jnp.swapaxes(ctx[:, :CTX_LEN, :], 0, 1)

    prompts_t = pl.pallas_call(
        _build_body,
        grid_spec=pltpu.PrefetchScalarGridSpec(
            num_scalar_prefetch=1,
            grid=(N_CLS,),
            in_specs=[
                pl.BlockSpec((1, 1, CTX_DIM), lambda c, nl: (c, 0, 0)),
                pl.BlockSpec((1, SUF_LEN, CTX_DIM), lambda c, nl: (c, 0, 0)),
                pl.BlockSpec((CTX_LEN, B, CTX_DIM), lambda c, nl: (0, 0, 0)),
            ],
            out_specs=pl.BlockSpec((SEQ_LEN, B, CTX_DIM),
                                   lambda c, nl: (0, c, 0)),
        ),
        out_shape=jax.ShapeDtypeStruct((SEQ_LEN, N_CLS * B, CTX_DIM),
                                       jnp.float32),
    )(name_lens, token_prefix, token_suffix, ctx_t)
    prompts = jnp.transpose(prompts_t, (1, 0, 2))

    tok = pl.pallas_call(
        _tok_body,
        grid=(B,),
        in_specs=[pl.BlockSpec((N_CLS, SEQ_LEN), lambda b: (0, 0))],
        out_specs=pl.BlockSpec((1, N_CLS, SEQ_LEN), lambda b: (b, 0, 0)),
        out_shape=jax.ShapeDtypeStruct((B, N_CLS, SEQ_LEN),
                                       tokenized_prompts.dtype),
    )(tokenized_prompts)

    return (prompts, tok.reshape(B * N_CLS, SEQ_LEN))


# speedup vs baseline: 3.6311x; 1.0199x over previous
import jax
import jax.numpy as jnp
from jax import lax
from jax.experimental import pallas as pl
from jax.experimental.pallas import tpu as pltpu
from jax.experimental.pallas import tpu_sc as plsc

B = 32
POOL = 1000
CTX_LEN = 36
CTX_DIM = 512
N_CLS = 100
SUF_LEN = 40
SEQ_LEN = 77
LANES = 16


def _sc_gather_body(idx_hbm, entity_hbm, out_hbm, idx_v, buf_v, sem):
    cid = lax.axis_index("c")
    sid = lax.axis_index("s")
    w = sid * 2 + cid
    pltpu.sync_copy(idx_hbm, idx_v.at[pl.ds(0, B)])
    my_idx = idx_v[pl.ds(w, LANES)][0]
    g = pl.multiple_of((my_idx // 8) * 8, 8)
    r = my_idx - g
    for off, n in ((0, 16), (16, 16), (32, 4)):
        n_pad = 8 if n == 4 else n
        pltpu.sync_copy(entity_hbm.at[pl.ds(off, n), pl.ds(g, 8)],
                        buf_v.at[pl.ds(0, n)])
        pltpu.sync_copy(buf_v.at[pl.ds(0, n_pad), r],
                        out_hbm.at[w, pl.ds(off, n_pad)])


CPB = 2


def _build_body(nl_ref, prefix_ref, suffix_ref, ctx_ref, out_ref):
    c = pl.program_id(0)
    pos = jax.lax.broadcasted_iota(jnp.int32, (SEQ_LEN, CTX_DIM), 0)
    for cc in range(CPB):
        nl = nl_ref[c * CPB + cc]
        s = suffix_ref[cc]
        p = prefix_ref[cc]
        s1 = jnp.concatenate([p, s, s[:SEQ_LEN - SUF_LEN - 1]], axis=0)
        s2 = jnp.concatenate([s[:SEQ_LEN - SUF_LEN], s], axis=0)
        base = jnp.where(pos <= nl, s1, s2)
        out_ref[:, cc * B:(cc + 1) * B, :] = jnp.broadcast_to(
            base[:, None, :], (SEQ_LEN, B, CTX_DIM))
        out_ref[pl.ds(nl + 1, CTX_LEN), cc * B:(cc + 1) * B, :] = ctx_ref[...]


def _tok_body(tok_ref, out_ref):
    out_ref[...] = tok_ref[...][None]


def kernel(indexs, entity_prompts, name_lens, token_prefix, token_suffix,
           tokenized_prompts, current_task):
    indexs = indexs.astype(jnp.int32)
    name_lens = name_lens.astype(jnp.int32)
    entity_t = jnp.transpose(entity_prompts, (1, 0, 2))

    mesh = plsc.VectorSubcoreMesh(core_axis_name="c", subcore_axis_name="s")
    ctx = pl.kernel(
        _sc_gather_body,
        out_type=jax.ShapeDtypeStruct((B, SUF_LEN, CTX_DIM), jnp.float32),
        mesh=mesh,
        scratch_types=[
            pltpu.VMEM((B + LANES,), jnp.int32),
            pltpu.VMEM((LANES, 8, CTX_DIM), jnp.float32),
            pltpu.SemaphoreType.DMA,
        ],
    )(indexs, entity_t)

    ctx_t = jnp.swapaxes(ctx[:, :CTX_LEN, :], 0, 1)

    prompts_t = pl.pallas_call(
        _build_body,
        grid_spec=pltpu.PrefetchScalarGridSpec(
            num_scalar_prefetch=1,
            grid=(N_CLS // CPB,),
            in_specs=[
                pl.BlockSpec((CPB, 1, CTX_DIM), lambda c, nl: (c, 0, 0)),
                pl.BlockSpec((CPB, SUF_LEN, CTX_DIM), lambda c, nl: (c, 0, 0)),
                pl.BlockSpec((CTX_LEN, B, CTX_DIM), lambda c, nl: (0, 0, 0)),
            ],
            out_specs=pl.BlockSpec((SEQ_LEN, CPB * B, CTX_DIM),
                                   lambda c, nl: (0, c, 0)),
        ),
        out_shape=jax.ShapeDtypeStruct((SEQ_LEN, N_CLS * B, CTX_DIM),
                                       jnp.float32),
    )(name_lens, token_prefix, token_suffix, ctx_t)
    prompts = jnp.transpose(prompts_t, (1, 0, 2))

    tok = pl.pallas_call(
        _tok_body,
        grid=(B,),
        in_specs=[pl.BlockSpec((N_CLS, SEQ_LEN), lambda b: (0, 0))],
        out_specs=pl.BlockSpec((1, N_CLS, SEQ_LEN), lambda b: (b, 0, 0)),
        out_shape=jax.ShapeDtypeStruct((B, N_CLS, SEQ_LEN),
                                       tokenized_prompts.dtype),
    )(tokenized_prompts)

    return (prompts, tok.reshape(B * N_CLS, SEQ_LEN))


# speedup vs baseline: 3.6358x; 1.0013x over previous
import jax
import jax.numpy as jnp
from jax import lax
from jax.experimental import pallas as pl
from jax.experimental.pallas import tpu as pltpu
from jax.experimental.pallas import tpu_sc as plsc

B = 32
POOL = 1000
CTX_LEN = 36
CTX_DIM = 512
N_CLS = 100
SUF_LEN = 40
SEQ_LEN = 77
LANES = 16


def _sc_gather_body(idx_hbm, entity_hbm, out_hbm, idx_v, buf_v, sem):
    cid = lax.axis_index("c")
    sid = lax.axis_index("s")
    w = sid * 2 + cid
    pltpu.sync_copy(idx_hbm, idx_v.at[pl.ds(0, B)])
    my_idx = idx_v[pl.ds(w, LANES)][0]
    g = pl.multiple_of((my_idx // 8) * 8, 8)
    r = my_idx - g
    for off, n in ((0, 16), (16, 16), (32, 4)):
        n_pad = 8 if n == 4 else n
        pltpu.sync_copy(entity_hbm.at[pl.ds(off, n), pl.ds(g, 8)],
                        buf_v.at[pl.ds(0, n)])
        pltpu.sync_copy(buf_v.at[pl.ds(0, n_pad), r],
                        out_hbm.at[w, pl.ds(off, n_pad)])


CPB = 4


def _build_body(nl_ref, prefix_ref, suffix_ref, ctx_ref, out_ref):
    c = pl.program_id(0)
    pos = jax.lax.broadcasted_iota(jnp.int32, (SEQ_LEN, CTX_DIM), 0)
    for cc in range(CPB):
        nl = nl_ref[c * CPB + cc]
        s = suffix_ref[cc]
        p = prefix_ref[cc]
        s1 = jnp.concatenate([p, s, s[:SEQ_LEN - SUF_LEN - 1]], axis=0)
        s2 = jnp.concatenate([s[:SEQ_LEN - SUF_LEN], s], axis=0)
        base = jnp.where(pos <= nl, s1, s2)
        out_ref[:, cc * B:(cc + 1) * B, :] = jnp.broadcast_to(
            base[:, None, :], (SEQ_LEN, B, CTX_DIM))
        out_ref[pl.ds(nl + 1, CTX_LEN), cc * B:(cc + 1) * B, :] = ctx_ref[...]


def _tok_body(tok_ref, out_ref):
    out_ref[...] = tok_ref[...][None]


def kernel(indexs, entity_prompts, name_lens, token_prefix, token_suffix,
           tokenized_prompts, current_task):
    indexs = indexs.astype(jnp.int32)
    name_lens = name_lens.astype(jnp.int32)
    entity_t = jnp.transpose(entity_prompts, (1, 0, 2))

    mesh = plsc.VectorSubcoreMesh(core_axis_name="c", subcore_axis_name="s")
    ctx = pl.kernel(
        _sc_gather_body,
        out_type=jax.ShapeDtypeStruct((B, SUF_LEN, CTX_DIM), jnp.float32),
        mesh=mesh,
        scratch_types=[
            pltpu.VMEM((B + LANES,), jnp.int32),
            pltpu.VMEM((LANES, 8, CTX_DIM), jnp.float32),
            pltpu.SemaphoreType.DMA,
        ],
    )(indexs, entity_t)

    ctx_t = jnp.swapaxes(ctx[:, :CTX_LEN, :], 0, 1)

    prompts_t = pl.pallas_call(
        _build_body,
        grid_spec=pltpu.PrefetchScalarGridSpec(
            num_scalar_prefetch=1,
            grid=(N_CLS // CPB,),
            in_specs=[
                pl.BlockSpec((CPB, 1, CTX_DIM), lambda c, nl: (c, 0, 0)),
                pl.BlockSpec((CPB, SUF_LEN, CTX_DIM), lambda c, nl: (c, 0, 0)),
                pl.BlockSpec((CTX_LEN, B, CTX_DIM), lambda c, nl: (0, 0, 0)),
            ],
            out_specs=pl.BlockSpec((SEQ_LEN, CPB * B, CTX_DIM),
                                   lambda c, nl: (0, c, 0)),
        ),
        out_shape=jax.ShapeDtypeStruct((SEQ_LEN, N_CLS * B, CTX_DIM),
                                       jnp.float32),
    )(name_lens, token_prefix, token_suffix, ctx_t)
    prompts = jnp.transpose(prompts_t, (1, 0, 2))

    tok = pl.pallas_call(
        _tok_body,
        grid=(B,),
        in_specs=[pl.BlockSpec((N_CLS, SEQ_LEN), lambda b: (0, 0))],
        out_specs=pl.BlockSpec((1, N_CLS, SEQ_LEN), lambda b: (b, 0, 0)),
        out_shape=jax.ShapeDtypeStruct((B, N_CLS, SEQ_LEN),
                                       tokenized_prompts.dtype),
    )(tokenized_prompts)

    return (prompts, tok.reshape(B * N_CLS, SEQ_LEN))
